# baseline (speedup 1.0000x reference)
import functools

import jax
import jax.numpy as jnp
from jax import lax
from jax.experimental import pallas as pl
from jax.experimental.pallas import tpu as pltpu
from jax.experimental.pallas import tpu_sc as plsc

NUM_CODES = 8192
CODE_DIM = 256
N_TOKENS = 8192

TN = 128
TK = 2048
TP = 256


def _sq_body(x_ref, c_ref, xsq_ref, csq_ref):
    xt = x_ref[...]
    ct = c_ref[...]
    xsq_ref[...] = jnp.sum(xt * xt, axis=1, keepdims=True)
    csq_ref[...] = jnp.sum(ct * ct, axis=1)[None, :]


def _sq_call(flat_x, codebook):
    n = flat_x.shape[0]
    return pl.pallas_call(
        _sq_body,
        grid=(n // TP,),
        in_specs=[
            pl.BlockSpec((TP, CODE_DIM), lambda i: (i, 0)),
            pl.BlockSpec((TP, CODE_DIM), lambda i: (i, 0)),
        ],
        out_specs=[
            pl.BlockSpec((TP, 1), lambda i: (i, 0)),
            pl.BlockSpec((1, TP), lambda i: (0, i)),
        ],
        out_shape=[
            jax.ShapeDtypeStruct((n, 1), jnp.float32),
            jax.ShapeDtypeStruct((1, NUM_CODES), jnp.float32),
        ],
    )(flat_x, codebook)


def _argmin_body(x_ref, c_ref, xsq_ref, csq_ref, idx_ref, minval, minidx):
    kk = pl.program_id(0)
    nn = pl.program_id(1)
    nk = pl.num_programs(0)
    rows = pl.ds(nn * TN, TN)
    xt = x_ref[...]
    ct = c_ref[...]
    x_sq = xsq_ref[...]
    mm2 = lax.dot_general(xt * -2.0, ct, (((1,), (1,)), ((), ())),
                          preferred_element_type=jnp.float32)
    acc = jnp.full((TN, 128), jnp.inf, dtype=jnp.float32)
    chunk_of = jnp.zeros((TN, 128), dtype=jnp.int32)
    for c in range(TK // 128):
        lo, hi = c * 128, (c + 1) * 128
        d2c = (x_sq + csq_ref[:, lo:hi]) + mm2[:, lo:hi]
        better = d2c < acc
        acc = jnp.minimum(acc, d2c)
        chunk_of = jnp.where(better, jnp.int32(c), chunk_of)
    tmin = jnp.min(acc, axis=1, keepdims=True)
    lane = lax.broadcasted_iota(jnp.int32, (TN, 128), 1)
    jidx = chunk_of * 128 + lane
    cand = jnp.where(acc == tmin, jidx, jnp.int32(2**30))
    tidx = (jnp.min(cand, axis=1, keepdims=True)
            + kk * TK)
    dmin = jnp.sqrt(jnp.maximum(tmin, 0.0))
    dmin_bf = dmin.astype(jnp.bfloat16).astype(jnp.float32)

    @pl.when(kk == 0)
    def _():
        minval[rows, :] = dmin_bf
        minidx[rows, :] = tidx

    @pl.when(kk > 0)
    def _():
        better = dmin < minval[rows, :]
        minval[rows, :] = jnp.where(better, dmin_bf, minval[rows, :])
        minidx[rows, :] = jnp.where(better, tidx, minidx[rows, :])

    @pl.when(kk == nk - 1)
    def _():
        idx_ref[...] = minidx[rows, :]


def _argmin_call(flat_x, codebook, xsq, csq):
    n, d = flat_x.shape
    kk = codebook.shape[0]
    grid = (kk // TK, n // TN)
    return pl.pallas_call(
        _argmin_body,
        grid=grid,
        in_specs=[
            pl.BlockSpec((TN, d), lambda k, i: (i, 0)),
            pl.BlockSpec((TK, d), lambda k, i: (k, 0)),
            pl.BlockSpec((TN, 1), lambda k, i: (i, 0)),
            pl.BlockSpec((1, TK), lambda k, i: (0, k)),
        ],
        out_specs=pl.BlockSpec((TN, 1), lambda k, i: (i, 0)),
        out_shape=jax.ShapeDtypeStruct((n, 1), jnp.int32),
        scratch_shapes=[
            pltpu.VMEM((n, 1), jnp.float32),
            pltpu.VMEM((n, 1), jnp.int32),
        ],
    )(flat_x, codebook, xsq, csq)


_NW = 32
_BPW = N_TOKENS // _NW


@functools.cache
def _make_sc_gather():
    @functools.partial(
        pl.kernel,
        out_type=jax.ShapeDtypeStruct((N_TOKENS, CODE_DIM), jnp.float32),
        mesh=plsc.VectorSubcoreMesh(core_axis_name="c", subcore_axis_name="s"),
        scratch_types=[
            pltpu.VMEM((_BPW,), jnp.int32),
            pltpu.VMEM((_BPW, CODE_DIM), jnp.float32),
            pltpu.SemaphoreType.DMA,
        ],
    )
    def _sc_gather(table_hbm, idx_hbm, out_hbm, idx_v, rows_v, sem):
        wid = lax.axis_index("s") * 2 + lax.axis_index("c")
        base = wid * _BPW
        pltpu.sync_copy(idx_hbm.at[pl.ds(base, _BPW)], idx_v)
        pltpu.async_copy(table_hbm.at[idx_v], rows_v, sem).wait()
        pltpu.sync_copy(rows_v, out_hbm.at[pl.ds(base, _BPW)])

    return _sc_gather


def kernel(x, codebook):
    b, s, d = x.shape
    flat_x = x.reshape(-1, d)
    xsq, csq = _sq_call(flat_x, codebook)
    idx = _argmin_call(flat_x, codebook, xsq, csq).reshape(-1)
    quantized = _make_sc_gather()(codebook, idx).reshape(b, s, d)
    return quantized, idx.reshape(b, s)


# speedup vs baseline: 1.9877x; 1.9877x over previous
import functools

import jax
import jax.numpy as jnp
from jax import lax
from jax.experimental import pallas as pl
from jax.experimental.pallas import tpu as pltpu
from jax.experimental.pallas import tpu_sc as plsc

NUM_CODES = 8192
CODE_DIM = 256
N_TOKENS = 8192

TN = 256
TK = 2048
TP = 256


def _sq_body(x_ref, c_ref, xsq_ref, csq_ref):
    xt = x_ref[...]
    ct = c_ref[...]
    xsq_ref[...] = jnp.sum(xt * xt, axis=1, keepdims=True)
    csq_ref[...] = jnp.sum(ct * ct, axis=1)[None, :]


def _sq_call(flat_x, codebook):
    n = flat_x.shape[0]
    return pl.pallas_call(
        _sq_body,
        grid=(n // TP,),
        in_specs=[
            pl.BlockSpec((TP, CODE_DIM), lambda i: (i, 0)),
            pl.BlockSpec((TP, CODE_DIM), lambda i: (i, 0)),
        ],
        out_specs=[
            pl.BlockSpec((TP, 1), lambda i: (i, 0)),
            pl.BlockSpec((1, TP), lambda i: (0, i)),
        ],
        out_shape=[
            jax.ShapeDtypeStruct((n, 1), jnp.float32),
            jax.ShapeDtypeStruct((1, NUM_CODES), jnp.float32),
        ],
    )(flat_x, codebook)


def _argmin_body(x_ref, c_ref, xsq_ref, csq_ref, idx_ref):
    xt = x_ref[...]
    x_sq = xsq_ref[...]
    xs2 = xt * -2.0
    best_val = None
    best_idx = None
    for w in range(NUM_CODES // TK):
        ct = c_ref[w * TK:(w + 1) * TK, :]
        mm2 = lax.dot_general(xs2, ct, (((1,), (1,)), ((), ())),
                              preferred_element_type=jnp.float32)
        acc = jnp.full((TN, 128), jnp.inf, dtype=jnp.float32)
        chunk_of = jnp.zeros((TN, 128), dtype=jnp.int32)
        for c in range(TK // 128):
            lo, hi = w * TK + c * 128, w * TK + (c + 1) * 128
            d2c = (x_sq + csq_ref[:, lo:hi]) + mm2[:, c * 128:(c + 1) * 128]
            better = d2c < acc
            acc = jnp.minimum(acc, d2c)
            chunk_of = jnp.where(better, jnp.int32(c), chunk_of)
        tmin = jnp.min(acc, axis=1, keepdims=True)
        lane = lax.broadcasted_iota(jnp.int32, (TN, 128), 1)
        jidx = chunk_of * 128 + lane
        cand = jnp.where(acc == tmin, jidx, jnp.int32(2**30))
        tidx = jnp.min(cand, axis=1, keepdims=True) + w * TK
        dmin = jnp.sqrt(jnp.maximum(tmin, 0.0))
        dmin_bf = dmin.astype(jnp.bfloat16).astype(jnp.float32)
        if w == 0:
            best_val, best_idx = dmin_bf, tidx
        else:
            take = dmin < best_val
            best_val = jnp.where(take, dmin_bf, best_val)
            best_idx = jnp.where(take, tidx, best_idx)
    idx_ref[...] = best_idx


def _argmin_call(flat_x, codebook, xsq, csq):
    n, d = flat_x.shape
    return pl.pallas_call(
        _argmin_body,
        grid=(n // TN,),
        in_specs=[
            pl.BlockSpec((TN, d), lambda i: (i, 0)),
            pl.BlockSpec((NUM_CODES, d), lambda i: (0, 0)),
            pl.BlockSpec((TN, 1), lambda i: (i, 0)),
            pl.BlockSpec((1, NUM_CODES), lambda i: (0, 0)),
        ],
        out_specs=pl.BlockSpec((TN, 1), lambda i: (i, 0)),
        out_shape=jax.ShapeDtypeStruct((n, 1), jnp.int32),
    )(flat_x, codebook, xsq, csq)


_NW = 32
_BPW = N_TOKENS // _NW


@functools.cache
def _make_sc_gather():
    @functools.partial(
        pl.kernel,
        out_type=jax.ShapeDtypeStruct((N_TOKENS, CODE_DIM), jnp.float32),
        mesh=plsc.VectorSubcoreMesh(core_axis_name="c", subcore_axis_name="s"),
        scratch_types=[
            pltpu.VMEM((_BPW,), jnp.int32),
            pltpu.VMEM((_BPW, CODE_DIM), jnp.float32),
            pltpu.SemaphoreType.DMA,
        ],
    )
    def _sc_gather(table_hbm, idx_hbm, out_hbm, idx_v, rows_v, sem):
        wid = lax.axis_index("s") * 2 + lax.axis_index("c")
        base = wid * _BPW
        pltpu.sync_copy(idx_hbm.at[pl.ds(base, _BPW)], idx_v)
        pltpu.async_copy(table_hbm.at[idx_v], rows_v, sem).wait()
        pltpu.sync_copy(rows_v, out_hbm.at[pl.ds(base, _BPW)])

    return _sc_gather


def kernel(x, codebook):
    b, s, d = x.shape
    flat_x = x.reshape(-1, d)
    xsq, csq = _sq_call(flat_x, codebook)
    idx = _argmin_call(flat_x, codebook, xsq, csq).reshape(-1)
    quantized = _make_sc_gather()(codebook, idx).reshape(b, s, d)
    return quantized, idx.reshape(b, s)
